# Initial kernel scaffold; baseline (speedup 1.0000x reference)
import jax
import jax.numpy as jnp
from jax.experimental import pallas as pl


def kernel(z, y, embedding_weight):
    raise NotImplementedError("write your pallas kernel here")


# speedup vs baseline: 1.0073x; 1.0073x over previous
import functools

import jax
import jax.numpy as jnp
from jax import lax
from jax.experimental import pallas as pl
from jax.experimental.pallas import tpu as pltpu
from jax.experimental.pallas import tpu_sc as plsc

B = 16384
D = 128
LANES = 16
NUM_WORKERS = 32
ROWS_PER_WORKER = B // NUM_WORKERS
CHUNK = 128
NCHUNKS = ROWS_PER_WORKER // CHUNK


def _body(z_hbm, y_hbm, w_hbm, out_hbm, idx_v, rows_v, z_v, sem_g, sem_z):
    wid = lax.axis_index("s") * 2 + lax.axis_index("c")
    base = wid * ROWS_PER_WORKER
    for ck in range(NCHUNKS):
        off = base + ck * CHUNK
        pltpu.sync_copy(y_hbm.at[pl.ds(off, CHUNK)], idx_v)
        gat = pltpu.async_copy(w_hbm.at[idx_v], rows_v, sem_g)
        zcp = pltpu.async_copy(z_hbm.at[pl.ds(off, CHUNK)], z_v, sem_z)
        gat.wait()
        zcp.wait()

        def add_row(r, carry):
            for c in range(D // LANES):
                s = pl.ds(c * LANES, LANES)
                rows_v[r, s] = rows_v[r, s] + z_v[r, s]
            return carry

        lax.fori_loop(0, CHUNK, add_row, 0)
        pltpu.sync_copy(rows_v, out_hbm.at[pl.ds(off, CHUNK)])


@jax.jit
def _run(z, y, embedding_weight):
    mesh = plsc.VectorSubcoreMesh(core_axis_name="c", subcore_axis_name="s")
    return pl.kernel(
        _body,
        out_type=jax.ShapeDtypeStruct((B, D), jnp.float32),
        mesh=mesh,
        scratch_types=[
            pltpu.VMEM((CHUNK,), jnp.int32),
            pltpu.VMEM((CHUNK, D), jnp.float32),
            pltpu.VMEM((CHUNK, D), jnp.float32),
            pltpu.SemaphoreType.DMA,
            pltpu.SemaphoreType.DMA,
        ],
    )(z, y, embedding_weight)


def kernel(z, y, embedding_weight):
    return _run(z, y.astype(jnp.int32), embedding_weight)


# speedup vs baseline: 1.2315x; 1.2225x over previous
import jax
import jax.numpy as jnp
from jax import lax
from jax.experimental import pallas as pl
from jax.experimental.pallas import tpu as pltpu
from jax.experimental.pallas import tpu_sc as plsc

B = 16384
D = 128
LANES = 16
NUM_WORKERS = 32
ROWS_PER_WORKER = B // NUM_WORKERS
CHUNK = 128
NCHUNKS = ROWS_PER_WORKER // CHUNK
NBUF = 3


def _body(z_hbm, y_hbm, w_hbm, out_hbm, idx_all, rows, zb, sem_g, sem_z, sem_o):
    wid = lax.axis_index("s") * 2 + lax.axis_index("c")
    base = wid * ROWS_PER_WORKER
    pltpu.sync_copy(y_hbm.at[wid], idx_all)

    def start_chunk(ck, p):
        g = pltpu.async_copy(w_hbm.at[idx_all.at[ck]], rows.at[p], sem_g.at[p])
        zc = pltpu.async_copy(
            z_hbm.at[pl.ds(base + ck * CHUNK, CHUNK)], zb.at[p], sem_z.at[p]
        )
        return g, zc

    inflight = {ck: start_chunk(ck, ck % NBUF) for ck in range(NBUF)}
    out_cps = {}
    for ck in range(NCHUNKS):
        p = ck % NBUF
        g, zc = inflight[ck]
        g.wait()
        zc.wait()

        @plsc.parallel_loop(0, CHUNK)
        def add_row(r):
            for c in range(D // LANES):
                s = pl.ds(c * LANES, LANES)
                zb[p, r, s] = zb[p, r, s] + rows[p, r, s]

        out_cps[ck] = pltpu.async_copy(
            zb.at[p], out_hbm.at[pl.ds(base + ck * CHUNK, CHUNK)], sem_o.at[p]
        )
        nxt = ck + NBUF
        if nxt < NCHUNKS:
            out_cps[ck].wait()
            inflight[nxt] = start_chunk(nxt, p)
    for ck in range(max(0, NCHUNKS - NBUF), NCHUNKS):
        out_cps[ck].wait()


@jax.jit
def _run(z, y, embedding_weight):
    mesh = plsc.VectorSubcoreMesh(core_axis_name="c", subcore_axis_name="s")
    return pl.kernel(
        _body,
        out_type=jax.ShapeDtypeStruct((B, D), jnp.float32),
        mesh=mesh,
        scratch_types=[
            pltpu.VMEM((NCHUNKS, CHUNK), jnp.int32),
            pltpu.VMEM((NBUF, CHUNK, D), jnp.float32),
            pltpu.VMEM((NBUF, CHUNK, D), jnp.float32),
            pltpu.SemaphoreType.DMA((NBUF,)),
            pltpu.SemaphoreType.DMA((NBUF,)),
            pltpu.SemaphoreType.DMA((NBUF,)),
        ],
    )(z, y.reshape(NUM_WORKERS, NCHUNKS, CHUNK), embedding_weight)


def kernel(z, y, embedding_weight):
    return _run(z, y.astype(jnp.int32), embedding_weight)
